# speedup vs baseline: 16.0911x; 16.0911x over previous
import functools

import jax
import jax.numpy as jnp
from jax import lax
from jax.experimental import pallas as pl
from jax.experimental.pallas import tpu as pltpu
from jax.experimental.pallas import tpu_sc as plsc

N = 10000
E = 320000
D = 128
NC = 2
NS = 16
NW = NC * NS
K = 128
EPAD = ((E + NW * K - 1) // (NW * K)) * (NW * K)
EPT = EPAD // NW
CHUNKS = EPT // K
APAD = 240
AROWS = N + APAD
RPT = AROWS // NS
RB = 1000

_mesh = plsc.VectorSubcoreMesh(core_axis_name="c", subcore_axis_name="s")


def _zero_vmem(buf, nrows, ncols):
    z16 = jnp.zeros((16,), jnp.float32)

    @pl.loop(0, nrows)
    def _(i):
        @pl.loop(0, ncols // 16)
        def _(j):
            buf[i, pl.ds(j * 16, 16)] = z16


def _deg_body(dst_hbm, out_hbm, acc, didx, upd, zbuf):
    c = lax.axis_index("c")
    s = lax.axis_index("s")
    wid = c * NS + s

    one0 = jnp.where(lax.iota(jnp.int32, 16) == 0, 1.0, 0.0)

    @pl.loop(0, K)
    def _(i):
        upd[i, pl.ds(0, 16)] = one0

    _zero_vmem(zbuf, K, 16)

    @pl.loop(0, RPT // K)
    def _(k):
        pltpu.sync_copy(zbuf, acc.at[pl.ds(s * RPT + k * K, K)])

    plsc.subcore_barrier()

    base = wid * EPT

    @pl.loop(0, CHUNKS)
    def _(t):
        pltpu.sync_copy(dst_hbm.at[pl.ds(base + t * K, K)], didx.at[0])
        pltpu.sync_copy(upd, acc.at[didx.at[0]], add=True)

    plsc.subcore_barrier()
    pltpu.sync_copy(acc.at[pl.ds(s * RPT, RPT)],
                    out_hbm.at[c, pl.ds(s * RPT, RPT)])


@jax.jit
def _sc_deg(dst_p):
    kfn = pl.kernel(
        _deg_body,
        out_type=jax.ShapeDtypeStruct((NC, AROWS, 16), jnp.float32),
        mesh=_mesh,
        scratch_types=[
            pltpu.VMEM_SHARED((AROWS, 16), jnp.float32),
            pltpu.VMEM((1, K), jnp.int32),
            pltpu.VMEM((K, 16), jnp.float32),
            pltpu.VMEM((K, 16), jnp.float32),
        ],
    )
    return kfn(dst_p)


def _agg_body(h_hbm, src_hbm, dst_hbm, out_hbm, acc, sidx, didx, rows):
    c = lax.axis_index("c")
    s = lax.axis_index("s")
    wid = c * NS + s

    _zero_vmem(rows, K, D)

    @pl.loop(0, RPT // K)
    def _(k):
        pltpu.sync_copy(rows, acc.at[pl.ds(s * RPT + k * K, K)])

    plsc.subcore_barrier()

    base = wid * EPT

    @pl.loop(0, CHUNKS)
    def _(t):
        pltpu.sync_copy(src_hbm.at[pl.ds(base + t * K, K)], sidx.at[0])
        pltpu.sync_copy(h_hbm.at[sidx.at[0]], rows)
        pltpu.sync_copy(dst_hbm.at[pl.ds(base + t * K, K)], didx.at[0])
        pltpu.sync_copy(rows, acc.at[didx.at[0]], add=True)

    plsc.subcore_barrier()
    pltpu.sync_copy(acc.at[pl.ds(s * RPT, RPT)],
                    out_hbm.at[c, pl.ds(s * RPT, RPT)])


@jax.jit
def _sc_agg(hp, src_p, dst_p):
    kfn = pl.kernel(
        _agg_body,
        out_type=jax.ShapeDtypeStruct((NC, AROWS, D), jnp.float32),
        mesh=_mesh,
        scratch_types=[
            pltpu.VMEM_SHARED((AROWS, D), jnp.float32),
            pltpu.VMEM((1, K), jnp.int32),
            pltpu.VMEM((1, K), jnp.int32),
            pltpu.VMEM((K, D), jnp.float32),
        ],
    )
    return kfn(hp, src_p, dst_p)


def _mm_body(x_ref, w_ref, o_ref):
    o_ref[...] = lax.dot_general(
        x_ref[...], w_ref[...], (((1,), (1,)), ((), ())),
        preferred_element_type=jnp.float32, precision=lax.Precision.HIGHEST)


@jax.jit
def _matmul(x, w):
    return pl.pallas_call(
        _mm_body,
        grid=(N // RB,),
        in_specs=[pl.BlockSpec((RB, D), lambda i: (i, 0)),
                  pl.BlockSpec((D, D), lambda i: (0, 0))],
        out_specs=pl.BlockSpec((RB, D), lambda i: (i, 0)),
        out_shape=jax.ShapeDtypeStruct((N, D), jnp.float32),
    )(x, w)


def _prep_body(parts_ref, h_ref, hp_ref, dis_ref):
    deg = parts_ref[0, :, 0:1] + parts_ref[1, :, 0:1] + 1.0
    dis = lax.rsqrt(deg)
    dis_ref[...] = dis
    hp_ref[...] = h_ref[...] * dis


@jax.jit
def _prep(parts, h):
    return pl.pallas_call(
        _prep_body,
        grid=(N // RB,),
        in_specs=[pl.BlockSpec((NC, RB, 16), lambda i: (0, i, 0)),
                  pl.BlockSpec((RB, D), lambda i: (i, 0))],
        out_specs=[pl.BlockSpec((RB, D), lambda i: (i, 0)),
                   pl.BlockSpec((RB, 1), lambda i: (i, 0))],
        out_shape=[jax.ShapeDtypeStruct((N, D), jnp.float32),
                   jax.ShapeDtypeStruct((N, 1), jnp.float32)],
    )(parts, h)


def _mid_body(p_ref, hp_ref, dis_ref, b_ref, w_ref, o_ref):
    t = dis_ref[...] * (p_ref[0] + p_ref[1] + hp_ref[...]) + b_ref[...]
    y = jnp.maximum(t, 0.0)
    h2 = lax.dot_general(
        y, w_ref[...], (((1,), (1,)), ((), ())),
        preferred_element_type=jnp.float32, precision=lax.Precision.HIGHEST)
    o_ref[...] = h2 * dis_ref[...]


@jax.jit
def _mid(parts, hp, dis, b, w):
    return pl.pallas_call(
        _mid_body,
        grid=(N // RB,),
        in_specs=[pl.BlockSpec((NC, RB, D), lambda i: (0, i, 0)),
                  pl.BlockSpec((RB, D), lambda i: (i, 0)),
                  pl.BlockSpec((RB, 1), lambda i: (i, 0)),
                  pl.BlockSpec((1, D), lambda i: (0, 0)),
                  pl.BlockSpec((D, D), lambda i: (0, 0))],
        out_specs=pl.BlockSpec((RB, D), lambda i: (i, 0)),
        out_shape=jax.ShapeDtypeStruct((N, D), jnp.float32),
    )(parts, hp, dis, b, w)


def _fin_body(p_ref, hp_ref, dis_ref, b_ref, o_ref):
    t = dis_ref[...] * (p_ref[0] + p_ref[1] + hp_ref[...]) + b_ref[...]
    m = jnp.max(t, axis=1, keepdims=True)
    lse = jnp.log(jnp.sum(jnp.exp(t - m), axis=1, keepdims=True)) + m
    o_ref[...] = t - lse


@jax.jit
def _fin(parts, hp, dis, b):
    return pl.pallas_call(
        _fin_body,
        grid=(N // RB,),
        in_specs=[pl.BlockSpec((NC, RB, D), lambda i: (0, i, 0)),
                  pl.BlockSpec((RB, D), lambda i: (i, 0)),
                  pl.BlockSpec((RB, 1), lambda i: (i, 0)),
                  pl.BlockSpec((1, D), lambda i: (0, 0))],
        out_specs=pl.BlockSpec((RB, D), lambda i: (i, 0)),
        out_shape=jax.ShapeDtypeStruct((N, D), jnp.float32),
    )(parts, hp, dis, b)


def kernel(x, edge_index, W1, b1, W2, b2):
    src = edge_index[0].astype(jnp.int32)
    dst = edge_index[1].astype(jnp.int32)
    npad = EPAD - E
    pad = jnp.arange(npad, dtype=jnp.int32)
    src_p = jnp.concatenate([src, pad % N])
    dst_p = jnp.concatenate([dst, N + pad % APAD])

    deg_parts = _sc_deg(dst_p)
    h1 = _matmul(x, W1)
    h1p, dis = _prep(deg_parts, h1)
    agg1 = _sc_agg(h1p, src_p, dst_p)
    h2p = _mid(agg1, h1p, dis, b1.reshape(1, D), W2)
    agg2 = _sc_agg(h2p, src_p, dst_p)
    return _fin(agg2, h2p, dis, b2.reshape(1, D))


# speedup vs baseline: 22.1594x; 1.3771x over previous
import functools

import jax
import jax.numpy as jnp
from jax import lax
from jax.experimental import pallas as pl
from jax.experimental.pallas import tpu as pltpu
from jax.experimental.pallas import tpu_sc as plsc

N = 10000
E = 320000
D = 128
NC = 2
NS = 16
NW = NC * NS
K = 256
EPAD = ((E + NW * K - 1) // (NW * K)) * (NW * K)
EPT = EPAD // NW
CHUNKS = EPT // K
ZB = 128
APAD = 240
AROWS = N + APAD
RPT = AROWS // NS
RB = 1000

_mesh = plsc.VectorSubcoreMesh(core_axis_name="c", subcore_axis_name="s")


def _zero_vmem(buf, nrows, ncols):
    z16 = jnp.zeros((16,), jnp.float32)

    @pl.loop(0, nrows)
    def _(i):
        @pl.loop(0, ncols // 16)
        def _(j):
            buf[i, pl.ds(j * 16, 16)] = z16


def _deg_body(dst_hbm, out_hbm, acc, didx, upd, zbuf):
    c = lax.axis_index("c")
    s = lax.axis_index("s")
    wid = c * NS + s

    one0 = jnp.where(lax.iota(jnp.int32, 16) == 0, 1.0, 0.0)

    @pl.loop(0, K)
    def _(i):
        upd[i, pl.ds(0, 16)] = one0

    _zero_vmem(zbuf, ZB, 16)

    @pl.loop(0, RPT // ZB)
    def _(k):
        pltpu.sync_copy(zbuf, acc.at[pl.ds(s * RPT + k * ZB, ZB)])

    plsc.subcore_barrier()

    base = wid * EPT

    @pl.loop(0, CHUNKS)
    def _(t):
        pltpu.sync_copy(dst_hbm.at[pl.ds(base + t * K, K)], didx.at[0])
        pltpu.sync_copy(upd, acc.at[didx.at[0]], add=True)

    plsc.subcore_barrier()
    pltpu.sync_copy(acc.at[pl.ds(s * RPT, RPT)],
                    out_hbm.at[c, pl.ds(s * RPT, RPT)])


@jax.jit
def _sc_deg(dst_p):
    kfn = pl.kernel(
        _deg_body,
        out_type=jax.ShapeDtypeStruct((NC, AROWS, 16), jnp.float32),
        mesh=_mesh,
        scratch_types=[
            pltpu.VMEM_SHARED((AROWS, 16), jnp.float32),
            pltpu.VMEM((1, K), jnp.int32),
            pltpu.VMEM((K, 16), jnp.float32),
            pltpu.VMEM((ZB, 16), jnp.float32),
        ],
    )
    return kfn(dst_p)


def _agg_body(h_hbm, src_hbm, dst_hbm, out_hbm, acc, sidx, didx, rows):
    c = lax.axis_index("c")
    s = lax.axis_index("s")
    wid = c * NS + s

    _zero_vmem(rows, ZB, D)

    @pl.loop(0, RPT // ZB)
    def _(k):
        pltpu.sync_copy(rows.at[pl.ds(0, ZB)],
                        acc.at[pl.ds(s * RPT + k * ZB, ZB)])

    base = wid * EPT
    pltpu.sync_copy(src_hbm.at[pl.ds(base, EPT)], sidx)

    plsc.subcore_barrier()

    @pl.loop(0, CHUNKS)
    def _(t):
        pltpu.sync_copy(h_hbm.at[sidx.at[pl.ds(t * K, K)]], rows)
        pltpu.sync_copy(dst_hbm.at[pl.ds(base + t * K, K)], didx.at[0])
        pltpu.sync_copy(rows, acc.at[didx.at[0]], add=True)

    plsc.subcore_barrier()
    pltpu.sync_copy(acc.at[pl.ds(s * RPT, RPT)],
                    out_hbm.at[c, pl.ds(s * RPT, RPT)])


@jax.jit
def _sc_agg(hp, src_p, dst_p):
    kfn = pl.kernel(
        _agg_body,
        out_type=jax.ShapeDtypeStruct((NC, AROWS, D), jnp.float32),
        mesh=_mesh,
        scratch_types=[
            pltpu.VMEM_SHARED((AROWS, D), jnp.float32),
            pltpu.VMEM((EPT,), jnp.int32),
            pltpu.VMEM((1, K), jnp.int32),
            pltpu.VMEM((K, D), jnp.float32),
        ],
    )
    return kfn(hp, src_p, dst_p)


def _mm_body(x_ref, w_ref, o_ref):
    o_ref[...] = lax.dot_general(
        x_ref[...], w_ref[...], (((1,), (1,)), ((), ())),
        preferred_element_type=jnp.float32, precision=lax.Precision.HIGHEST)


@jax.jit
def _matmul(x, w):
    return pl.pallas_call(
        _mm_body,
        grid=(N // RB,),
        in_specs=[pl.BlockSpec((RB, D), lambda i: (i, 0)),
                  pl.BlockSpec((D, D), lambda i: (0, 0))],
        out_specs=pl.BlockSpec((RB, D), lambda i: (i, 0)),
        out_shape=jax.ShapeDtypeStruct((N, D), jnp.float32),
    )(x, w)


def _prep_body(parts_ref, h_ref, hp_ref, dis_ref):
    deg = parts_ref[0, :, 0:1] + parts_ref[1, :, 0:1] + 1.0
    dis = lax.rsqrt(deg)
    dis_ref[...] = dis
    hp_ref[...] = h_ref[...] * dis


@jax.jit
def _prep(parts, h):
    return pl.pallas_call(
        _prep_body,
        grid=(N // RB,),
        in_specs=[pl.BlockSpec((NC, RB, 16), lambda i: (0, i, 0)),
                  pl.BlockSpec((RB, D), lambda i: (i, 0))],
        out_specs=[pl.BlockSpec((RB, D), lambda i: (i, 0)),
                   pl.BlockSpec((RB, 1), lambda i: (i, 0))],
        out_shape=[jax.ShapeDtypeStruct((N, D), jnp.float32),
                   jax.ShapeDtypeStruct((N, 1), jnp.float32)],
    )(parts, h)


def _mid_body(p_ref, hp_ref, dis_ref, b_ref, w_ref, o_ref):
    t = dis_ref[...] * (p_ref[0] + p_ref[1] + hp_ref[...]) + b_ref[...]
    y = jnp.maximum(t, 0.0)
    h2 = lax.dot_general(
        y, w_ref[...], (((1,), (1,)), ((), ())),
        preferred_element_type=jnp.float32, precision=lax.Precision.HIGHEST)
    o_ref[...] = h2 * dis_ref[...]


@jax.jit
def _mid(parts, hp, dis, b, w):
    return pl.pallas_call(
        _mid_body,
        grid=(N // RB,),
        in_specs=[pl.BlockSpec((NC, RB, D), lambda i: (0, i, 0)),
                  pl.BlockSpec((RB, D), lambda i: (i, 0)),
                  pl.BlockSpec((RB, 1), lambda i: (i, 0)),
                  pl.BlockSpec((1, D), lambda i: (0, 0)),
                  pl.BlockSpec((D, D), lambda i: (0, 0))],
        out_specs=pl.BlockSpec((RB, D), lambda i: (i, 0)),
        out_shape=jax.ShapeDtypeStruct((N, D), jnp.float32),
    )(parts, hp, dis, b, w)


def _fin_body(p_ref, hp_ref, dis_ref, b_ref, o_ref):
    t = dis_ref[...] * (p_ref[0] + p_ref[1] + hp_ref[...]) + b_ref[...]
    m = jnp.max(t, axis=1, keepdims=True)
    lse = jnp.log(jnp.sum(jnp.exp(t - m), axis=1, keepdims=True)) + m
    o_ref[...] = t - lse


@jax.jit
def _fin(parts, hp, dis, b):
    return pl.pallas_call(
        _fin_body,
        grid=(N // RB,),
        in_specs=[pl.BlockSpec((NC, RB, D), lambda i: (0, i, 0)),
                  pl.BlockSpec((RB, D), lambda i: (i, 0)),
                  pl.BlockSpec((RB, 1), lambda i: (i, 0)),
                  pl.BlockSpec((1, D), lambda i: (0, 0))],
        out_specs=pl.BlockSpec((RB, D), lambda i: (i, 0)),
        out_shape=jax.ShapeDtypeStruct((N, D), jnp.float32),
    )(parts, hp, dis, b)


def kernel(x, edge_index, W1, b1, W2, b2):
    src = edge_index[0].astype(jnp.int32)
    dst = edge_index[1].astype(jnp.int32)
    npad = EPAD - E
    pad = jnp.arange(npad, dtype=jnp.int32)
    src_p = jnp.concatenate([src, pad % N])
    dst_p = jnp.concatenate([dst, N + pad % APAD])

    deg_parts = _sc_deg(dst_p)
    h1 = _matmul(x, W1)
    h1p, dis = _prep(deg_parts, h1)
    agg1 = _sc_agg(h1p, src_p, dst_p)
    h2p = _mid(agg1, h1p, dis, b1.reshape(1, D), W2)
    agg2 = _sc_agg(h2p, src_p, dst_p)
    return _fin(agg2, h2p, dis, b2.reshape(1, D))


# speedup vs baseline: 22.7595x; 1.0271x over previous
import functools

import jax
import jax.numpy as jnp
from jax import lax
from jax.experimental import pallas as pl
from jax.experimental.pallas import tpu as pltpu
from jax.experimental.pallas import tpu_sc as plsc

N = 10000
E = 320000
D = 128
NC = 2
NS = 16
NW = NC * NS
K = 256
KA = 128
EPAD = ((E + NW * K - 1) // (NW * K)) * (NW * K)
EPT = EPAD // NW
CHUNKS = EPT // K
CHA = EPT // KA
ZB = 128
APAD = 240
AROWS = N + APAD
RPT = AROWS // NS
RB = 1000

_mesh = plsc.VectorSubcoreMesh(core_axis_name="c", subcore_axis_name="s")


def _zero_vmem(buf, nrows, ncols):
    z16 = jnp.zeros((16,), jnp.float32)

    @pl.loop(0, nrows)
    def _(i):
        @pl.loop(0, ncols // 16)
        def _(j):
            buf[i, pl.ds(j * 16, 16)] = z16


def _deg_body(dst_hbm, out_hbm, acc, didx, upd, zbuf):
    c = lax.axis_index("c")
    s = lax.axis_index("s")
    wid = c * NS + s

    one0 = jnp.where(lax.iota(jnp.int32, 16) == 0, 1.0, 0.0)

    @pl.loop(0, K)
    def _(i):
        upd[i, pl.ds(0, 16)] = one0

    _zero_vmem(zbuf, ZB, 16)

    @pl.loop(0, RPT // ZB)
    def _(k):
        pltpu.sync_copy(zbuf, acc.at[pl.ds(s * RPT + k * ZB, ZB)])

    plsc.subcore_barrier()

    base = wid * EPT

    @pl.loop(0, CHUNKS)
    def _(t):
        pltpu.sync_copy(dst_hbm.at[pl.ds(base + t * K, K)], didx.at[0])
        pltpu.sync_copy(upd, acc.at[didx.at[0]], add=True)

    plsc.subcore_barrier()
    pltpu.sync_copy(acc.at[pl.ds(s * RPT, RPT)],
                    out_hbm.at[c, pl.ds(s * RPT, RPT)])


@jax.jit
def _sc_deg(dst_p):
    kfn = pl.kernel(
        _deg_body,
        out_type=jax.ShapeDtypeStruct((NC, AROWS, 16), jnp.float32),
        mesh=_mesh,
        scratch_types=[
            pltpu.VMEM_SHARED((AROWS, 16), jnp.float32),
            pltpu.VMEM((1, K), jnp.int32),
            pltpu.VMEM((K, 16), jnp.float32),
            pltpu.VMEM((ZB, 16), jnp.float32),
        ],
    )
    return kfn(dst_p)


def _agg_body(h_hbm, src_hbm, dst_hbm, out_hbm, acc, sidx,
              didx0, didx1, rows0, rows1, gsem0, gsem1, dsem0, dsem1):
    c = lax.axis_index("c")
    s = lax.axis_index("s")
    wid = c * NS + s

    _zero_vmem(rows0, ZB, D)

    @pl.loop(0, RPT // ZB)
    def _(k):
        pltpu.sync_copy(rows0.at[pl.ds(0, ZB)],
                        acc.at[pl.ds(s * RPT + k * ZB, ZB)])

    base = wid * EPT
    pltpu.sync_copy(src_hbm.at[pl.ds(base, EPT)], sidx)

    plsc.subcore_barrier()

    bufs = ((rows0, gsem0, didx0, dsem0), (rows1, gsem1, didx1, dsem1))
    for b in range(2):
        rows_b, ssem, didx_b, _ = bufs[b]
        pltpu.sync_copy(dst_hbm.at[pl.ds(base + b * KA, KA)], didx_b.at[0])
        pltpu.sync_copy(h_hbm.at[sidx.at[pl.ds(b * KA, KA)]], rows_b)
        pltpu.async_copy(rows_b, acc.at[didx_b.at[0]], ssem, add=True)

    @pl.loop(2, CHA, step=2)
    def _(t):
        for b in range(2):
            rows_b, ssem, didx_b, _ = bufs[b]
            tb = t + b
            pltpu.make_async_copy(
                rows_b, acc.at[didx_b.at[0]], ssem).wait()
            pltpu.sync_copy(dst_hbm.at[pl.ds(base + tb * KA, KA)],
                            didx_b.at[0])
            pltpu.sync_copy(h_hbm.at[sidx.at[pl.ds(tb * KA, KA)]], rows_b)
            pltpu.async_copy(rows_b, acc.at[didx_b.at[0]], ssem, add=True)

    for b in range(2):
        rows_b, ssem, didx_b, _ = bufs[b]
        pltpu.make_async_copy(rows_b, acc.at[didx_b.at[0]], ssem).wait()

    plsc.subcore_barrier()
    pltpu.sync_copy(acc.at[pl.ds(s * RPT, RPT)],
                    out_hbm.at[c, pl.ds(s * RPT, RPT)])


@jax.jit
def _sc_agg(hp, src_p, dst_p):
    kfn = pl.kernel(
        _agg_body,
        out_type=jax.ShapeDtypeStruct((NC, AROWS, D), jnp.float32),
        mesh=_mesh,
        scratch_types=[
            pltpu.VMEM_SHARED((AROWS, D), jnp.float32),
            pltpu.VMEM((EPT,), jnp.int32),
            pltpu.VMEM((1, KA), jnp.int32),
            pltpu.VMEM((1, KA), jnp.int32),
            pltpu.VMEM((KA, D), jnp.float32),
            pltpu.VMEM((KA, D), jnp.float32),
            pltpu.SemaphoreType.DMA,
            pltpu.SemaphoreType.DMA,
            pltpu.SemaphoreType.DMA,
            pltpu.SemaphoreType.DMA,
        ],
    )
    return kfn(hp, src_p, dst_p)


def _mm_body(x_ref, w_ref, o_ref):
    o_ref[...] = lax.dot_general(
        x_ref[...], w_ref[...], (((1,), (1,)), ((), ())),
        preferred_element_type=jnp.float32, precision=lax.Precision.HIGHEST)


@jax.jit
def _matmul(x, w):
    return pl.pallas_call(
        _mm_body,
        grid=(N // RB,),
        in_specs=[pl.BlockSpec((RB, D), lambda i: (i, 0)),
                  pl.BlockSpec((D, D), lambda i: (0, 0))],
        out_specs=pl.BlockSpec((RB, D), lambda i: (i, 0)),
        out_shape=jax.ShapeDtypeStruct((N, D), jnp.float32),
    )(x, w)


def _prep_body(parts_ref, h_ref, hp_ref, dis_ref):
    deg = parts_ref[0, :, 0:1] + parts_ref[1, :, 0:1] + 1.0
    dis = lax.rsqrt(deg)
    dis_ref[...] = dis
    hp_ref[...] = h_ref[...] * dis


@jax.jit
def _prep(parts, h):
    return pl.pallas_call(
        _prep_body,
        grid=(N // RB,),
        in_specs=[pl.BlockSpec((NC, RB, 16), lambda i: (0, i, 0)),
                  pl.BlockSpec((RB, D), lambda i: (i, 0))],
        out_specs=[pl.BlockSpec((RB, D), lambda i: (i, 0)),
                   pl.BlockSpec((RB, 1), lambda i: (i, 0))],
        out_shape=[jax.ShapeDtypeStruct((N, D), jnp.float32),
                   jax.ShapeDtypeStruct((N, 1), jnp.float32)],
    )(parts, h)


def _mid_body(p_ref, hp_ref, dis_ref, b_ref, w_ref, o_ref):
    t = dis_ref[...] * (p_ref[0] + p_ref[1] + hp_ref[...]) + b_ref[...]
    y = jnp.maximum(t, 0.0)
    h2 = lax.dot_general(
        y, w_ref[...], (((1,), (1,)), ((), ())),
        preferred_element_type=jnp.float32, precision=lax.Precision.HIGHEST)
    o_ref[...] = h2 * dis_ref[...]


@jax.jit
def _mid(parts, hp, dis, b, w):
    return pl.pallas_call(
        _mid_body,
        grid=(N // RB,),
        in_specs=[pl.BlockSpec((NC, RB, D), lambda i: (0, i, 0)),
                  pl.BlockSpec((RB, D), lambda i: (i, 0)),
                  pl.BlockSpec((RB, 1), lambda i: (i, 0)),
                  pl.BlockSpec((1, D), lambda i: (0, 0)),
                  pl.BlockSpec((D, D), lambda i: (0, 0))],
        out_specs=pl.BlockSpec((RB, D), lambda i: (i, 0)),
        out_shape=jax.ShapeDtypeStruct((N, D), jnp.float32),
    )(parts, hp, dis, b, w)


def _fin_body(p_ref, hp_ref, dis_ref, b_ref, o_ref):
    t = dis_ref[...] * (p_ref[0] + p_ref[1] + hp_ref[...]) + b_ref[...]
    m = jnp.max(t, axis=1, keepdims=True)
    lse = jnp.log(jnp.sum(jnp.exp(t - m), axis=1, keepdims=True)) + m
    o_ref[...] = t - lse


@jax.jit
def _fin(parts, hp, dis, b):
    return pl.pallas_call(
        _fin_body,
        grid=(N // RB,),
        in_specs=[pl.BlockSpec((NC, RB, D), lambda i: (0, i, 0)),
                  pl.BlockSpec((RB, D), lambda i: (i, 0)),
                  pl.BlockSpec((RB, 1), lambda i: (i, 0)),
                  pl.BlockSpec((1, D), lambda i: (0, 0))],
        out_specs=pl.BlockSpec((RB, D), lambda i: (i, 0)),
        out_shape=jax.ShapeDtypeStruct((N, D), jnp.float32),
    )(parts, hp, dis, b)


def kernel(x, edge_index, W1, b1, W2, b2):
    src = edge_index[0].astype(jnp.int32)
    dst = edge_index[1].astype(jnp.int32)
    npad = EPAD - E
    pad = jnp.arange(npad, dtype=jnp.int32)
    src_p = jnp.concatenate([src, pad % N])
    dst_p = jnp.concatenate([dst, N + pad % APAD])

    deg_parts = _sc_deg(dst_p)
    h1 = _matmul(x, W1)
    h1p, dis = _prep(deg_parts, h1)
    agg1 = _sc_agg(h1p, src_p, dst_p)
    h2p = _mid(agg1, h1p, dis, b1.reshape(1, D), W2)
    agg2 = _sc_agg(h2p, src_p, dst_p)
    return _fin(agg2, h2p, dis, b2.reshape(1, D))


# speedup vs baseline: 26.8684x; 1.1805x over previous
import functools

import jax
import jax.numpy as jnp
from jax import lax
from jax.experimental import pallas as pl
from jax.experimental.pallas import tpu as pltpu
from jax.experimental.pallas import tpu_sc as plsc

N = 10000
E = 320000
D = 128
NC = 2
NS = 16
NW = NC * NS
K = 256
KA = 128
EPAD = ((E + NW * K - 1) // (NW * K)) * (NW * K)
EPT = EPAD // NW
CHUNKS = EPT // K
CHA = EPT // KA
ZB = 128
APAD = 240
AROWS = N + APAD
RPT = AROWS // NS
RB = 1000

_mesh = plsc.VectorSubcoreMesh(core_axis_name="c", subcore_axis_name="s")


def _zero_vmem(buf, nrows, ncols):
    z16 = jnp.zeros((16,), jnp.float32)

    @pl.loop(0, nrows)
    def _(i):
        @pl.loop(0, ncols // 16)
        def _(j):
            buf[i, pl.ds(j * 16, 16)] = z16


def _deg_body(dst_hbm, out_hbm, acc, didx, upd, zbuf):
    c = lax.axis_index("c")
    s = lax.axis_index("s")
    wid = c * NS + s

    one0 = jnp.where(lax.iota(jnp.int32, 16) == 0, 1.0, 0.0)

    @pl.loop(0, K)
    def _(i):
        upd[i, pl.ds(0, 16)] = one0

    _zero_vmem(zbuf, ZB, 16)

    @pl.loop(0, RPT // ZB)
    def _(k):
        pltpu.sync_copy(zbuf, acc.at[pl.ds(s * RPT + k * ZB, ZB)])

    plsc.subcore_barrier()

    base = wid * EPT

    @pl.loop(0, CHUNKS)
    def _(t):
        pltpu.sync_copy(dst_hbm.at[pl.ds(base + t * K, K)], didx.at[0])
        pltpu.sync_copy(upd, acc.at[didx.at[0]], add=True)

    plsc.subcore_barrier()
    pltpu.sync_copy(acc.at[pl.ds(s * RPT, RPT)],
                    out_hbm.at[c, pl.ds(s * RPT, RPT)])


@jax.jit
def _sc_deg(dst_p):
    kfn = pl.kernel(
        _deg_body,
        out_type=jax.ShapeDtypeStruct((NC, AROWS, 16), jnp.float32),
        mesh=_mesh,
        scratch_types=[
            pltpu.VMEM_SHARED((AROWS, 16), jnp.float32),
            pltpu.VMEM((1, K), jnp.int32),
            pltpu.VMEM((K, 16), jnp.float32),
            pltpu.VMEM((ZB, 16), jnp.float32),
        ],
    )
    return kfn(dst_p)


HEPT = EPT // 2
HALF = CHA // 2


def _agg_body(h_hbm, src_hbm, dst3_hbm, out_hbm, acc, sidx, didx3,
              rows0, rows1, ssem0, ssem1):
    c = lax.axis_index("c")
    s = lax.axis_index("s")
    wid = c * NS + s

    _zero_vmem(rows0, ZB, D)

    @pl.loop(0, RPT // ZB)
    def _(k):
        pltpu.sync_copy(rows0.at[pl.ds(0, ZB)],
                        acc.at[pl.ds(s * RPT + k * ZB, ZB)])

    base = wid * EPT
    pltpu.sync_copy(dst3_hbm.at[wid], didx3)
    pltpu.sync_copy(src_hbm.at[pl.ds(base, HEPT)], sidx)

    plsc.subcore_barrier()

    bufs = ((rows0, ssem0), (rows1, ssem1))
    for b in range(2):
        rows_b, ssem = bufs[b]
        pltpu.sync_copy(h_hbm.at[sidx.at[pl.ds(b * KA, KA)]], rows_b)
        pltpu.async_copy(rows_b, acc.at[didx3.at[b]], ssem, add=True)

    def _steady(t, b, half):
        rows_b, ssem = bufs[b]
        tb = t + b
        pltpu.make_async_copy(rows_b, acc.at[didx3.at[tb - 2]], ssem).wait()
        pltpu.sync_copy(
            h_hbm.at[sidx.at[pl.ds((tb - half * HALF) * KA, KA)]], rows_b)
        pltpu.async_copy(rows_b, acc.at[didx3.at[tb]], ssem, add=True)

    @pl.loop(2, HALF, step=2)
    def _(t):
        for b in range(2):
            _steady(t, b, 0)

    pltpu.sync_copy(src_hbm.at[pl.ds(base + HEPT, HEPT)], sidx)

    @pl.loop(HALF, CHA, step=2)
    def _(t):
        for b in range(2):
            _steady(t, b, 1)

    for b in range(2):
        rows_b, ssem = bufs[b]
        pltpu.make_async_copy(
            rows_b, acc.at[didx3.at[CHA - 2 + b]], ssem).wait()

    plsc.subcore_barrier()
    pltpu.sync_copy(acc.at[pl.ds(s * RPT, RPT)],
                    out_hbm.at[c, pl.ds(s * RPT, RPT)])


@jax.jit
def _sc_agg(hp, src_p, dst3_p):
    kfn = pl.kernel(
        _agg_body,
        out_type=jax.ShapeDtypeStruct((NC, AROWS, D), jnp.float32),
        mesh=_mesh,
        scratch_types=[
            pltpu.VMEM_SHARED((AROWS, D), jnp.float32),
            pltpu.VMEM((HEPT,), jnp.int32),
            pltpu.VMEM((CHA, KA), jnp.int32),
            pltpu.VMEM((KA, D), jnp.float32),
            pltpu.VMEM((KA, D), jnp.float32),
            pltpu.SemaphoreType.DMA,
            pltpu.SemaphoreType.DMA,
        ],
    )
    return kfn(hp, src_p, dst3_p)


def _mm_body(x_ref, w_ref, o_ref):
    o_ref[...] = lax.dot_general(
        x_ref[...], w_ref[...], (((1,), (1,)), ((), ())),
        preferred_element_type=jnp.float32, precision=lax.Precision.HIGHEST)


@jax.jit
def _matmul(x, w):
    return pl.pallas_call(
        _mm_body,
        grid=(N // RB,),
        in_specs=[pl.BlockSpec((RB, D), lambda i: (i, 0)),
                  pl.BlockSpec((D, D), lambda i: (0, 0))],
        out_specs=pl.BlockSpec((RB, D), lambda i: (i, 0)),
        out_shape=jax.ShapeDtypeStruct((N, D), jnp.float32),
    )(x, w)


def _prep_body(parts_ref, h_ref, hp_ref, dis_ref):
    deg = parts_ref[0, :, 0:1] + parts_ref[1, :, 0:1] + 1.0
    dis = lax.rsqrt(deg)
    dis_ref[...] = dis
    hp_ref[...] = h_ref[...] * dis


@jax.jit
def _prep(parts, h):
    return pl.pallas_call(
        _prep_body,
        grid=(N // RB,),
        in_specs=[pl.BlockSpec((NC, RB, 16), lambda i: (0, i, 0)),
                  pl.BlockSpec((RB, D), lambda i: (i, 0))],
        out_specs=[pl.BlockSpec((RB, D), lambda i: (i, 0)),
                   pl.BlockSpec((RB, 1), lambda i: (i, 0))],
        out_shape=[jax.ShapeDtypeStruct((N, D), jnp.float32),
                   jax.ShapeDtypeStruct((N, 1), jnp.float32)],
    )(parts, h)


def _mid_body(p_ref, hp_ref, dis_ref, b_ref, w_ref, o_ref):
    t = dis_ref[...] * (p_ref[0] + p_ref[1] + hp_ref[...]) + b_ref[...]
    y = jnp.maximum(t, 0.0)
    h2 = lax.dot_general(
        y, w_ref[...], (((1,), (1,)), ((), ())),
        preferred_element_type=jnp.float32, precision=lax.Precision.HIGHEST)
    o_ref[...] = h2 * dis_ref[...]


@jax.jit
def _mid(parts, hp, dis, b, w):
    return pl.pallas_call(
        _mid_body,
        grid=(N // RB,),
        in_specs=[pl.BlockSpec((NC, RB, D), lambda i: (0, i, 0)),
                  pl.BlockSpec((RB, D), lambda i: (i, 0)),
                  pl.BlockSpec((RB, 1), lambda i: (i, 0)),
                  pl.BlockSpec((1, D), lambda i: (0, 0)),
                  pl.BlockSpec((D, D), lambda i: (0, 0))],
        out_specs=pl.BlockSpec((RB, D), lambda i: (i, 0)),
        out_shape=jax.ShapeDtypeStruct((N, D), jnp.float32),
    )(parts, hp, dis, b, w)


def _fin_body(p_ref, hp_ref, dis_ref, b_ref, o_ref):
    t = dis_ref[...] * (p_ref[0] + p_ref[1] + hp_ref[...]) + b_ref[...]
    m = jnp.max(t, axis=1, keepdims=True)
    lse = jnp.log(jnp.sum(jnp.exp(t - m), axis=1, keepdims=True)) + m
    o_ref[...] = t - lse


@jax.jit
def _fin(parts, hp, dis, b):
    return pl.pallas_call(
        _fin_body,
        grid=(N // RB,),
        in_specs=[pl.BlockSpec((NC, RB, D), lambda i: (0, i, 0)),
                  pl.BlockSpec((RB, D), lambda i: (i, 0)),
                  pl.BlockSpec((RB, 1), lambda i: (i, 0)),
                  pl.BlockSpec((1, D), lambda i: (0, 0))],
        out_specs=pl.BlockSpec((RB, D), lambda i: (i, 0)),
        out_shape=jax.ShapeDtypeStruct((N, D), jnp.float32),
    )(parts, hp, dis, b)


def kernel(x, edge_index, W1, b1, W2, b2):
    src = edge_index[0].astype(jnp.int32)
    dst = edge_index[1].astype(jnp.int32)
    npad = EPAD - E
    pad = jnp.arange(npad, dtype=jnp.int32)
    src_p = jnp.concatenate([src, pad % N])
    dst_p = jnp.concatenate([dst, N + pad % APAD])

    dst3_p = dst_p.reshape(NW, CHA, KA)

    deg_parts = _sc_deg(dst_p)
    h1 = _matmul(x, W1)
    h1p, dis = _prep(deg_parts, h1)
    agg1 = _sc_agg(h1p, src_p, dst3_p)
    h2p = _mid(agg1, h1p, dis, b1.reshape(1, D), W2)
    agg2 = _sc_agg(h2p, src_p, dst3_p)
    return _fin(agg2, h2p, dis, b2.reshape(1, D))
